# baseline (speedup 1.0000x reference)
import jax
import jax.numpy as jnp
from jax.experimental import pallas as pl
from jax.experimental.pallas import tpu as pltpu

_NF4_VALS = (
    -1.0, -0.6961928009986877, -0.5250730514526367, -0.39491748809814453,
    -0.28444138169288635, -0.18477343022823334, -0.09105003625154495, 0.0,
    0.07958029955625534, 0.16093020141124725, 0.24611230194568634,
    0.33791524171829224, 0.44070982933044434, 0.5626170039176941,
    0.7229568362236023, 1.0)

_QBLOCK = 64
_LORA_SCALE = 2.0
_NSLAB = 16
_QDEPTH = 4


def _fused_kernel(cb_ref, scales_ref, sexp_ref, lora_a_ref, lora_b_ref,
                  w_idx_ref, x_ref, y_ref, w_eff_ref, wbuf_ref, sem_ref):
    i = pl.program_id(0)

    @pl.when(i == 0)
    def _prep():
        o, d = w_eff_ref.shape
        rows = o // _NSLAB

        def copy(j):
            return pltpu.make_async_copy(
                w_idx_ref.at[pl.ds(j * rows, rows), :],
                wbuf_ref.at[j % _QDEPTH], sem_ref.at[j % _QDEPTH])

        for j in range(_QDEPTH):
            copy(j).start()
        for j in range(_NSLAB):
            copy(j).wait()
            idx = wbuf_ref[j % _QDEPTH]
            cb = jnp.broadcast_to(cb_ref[...], (rows, 16))
            deq = jnp.take_along_axis(cb, idx, axis=1)
            sc = jax.lax.dot_general(
                scales_ref[pl.ds(j * rows, rows), :], sexp_ref[...],
                (((1,), (0,)), ((), ())), preferred_element_type=jnp.float32)
            lora = jax.lax.dot_general(
                lora_b_ref[pl.ds(j * rows, rows), :], lora_a_ref[...],
                (((1,), (0,)), ((), ())), preferred_element_type=jnp.float32)
            w_eff_ref[pl.ds(j * rows, rows), :] = (
                deq * sc + _LORA_SCALE * lora).astype(jnp.bfloat16)
            if j + _QDEPTH < _NSLAB:
                copy(j + _QDEPTH).start()

    @pl.when(i > 0)
    def _mm():
        xb = x_ref[...].astype(jnp.bfloat16)
        y_ref[...] = jax.lax.dot_general(
            xb, w_eff_ref[...], (((1,), (1,)), ((), ())),
            preferred_element_type=jnp.float32)


def kernel(x, w_idx, scales, lora_a, lora_b):
    b, s, d = x.shape
    o = w_idx.shape[0]
    m = b * s
    x2 = x.reshape(m, d)

    mt = 2048
    nsteps = m // mt

    def xmap(i):
        return (jnp.maximum(i - 1, 0), 0)

    y2 = pl.pallas_call(
        _fused_kernel,
        grid=(nsteps + 1,),
        in_specs=[
            pl.BlockSpec((1, 16), lambda i: (0, 0)),
            pl.BlockSpec((o, scales.shape[1]), lambda i: (0, 0)),
            pl.BlockSpec((scales.shape[1], d), lambda i: (0, 0)),
            pl.BlockSpec(lora_a.shape, lambda i: (0, 0)),
            pl.BlockSpec((o, lora_b.shape[1]), lambda i: (0, 0)),
            pl.BlockSpec(memory_space=pl.ANY),
            pl.BlockSpec((mt, d), xmap),
        ],
        out_specs=pl.BlockSpec((mt, o), xmap),
        out_shape=jax.ShapeDtypeStruct((m, o), jnp.float32),
        scratch_shapes=[
            pltpu.VMEM((o, d), jnp.bfloat16),
            pltpu.VMEM((_QDEPTH, o // _NSLAB, d), jnp.int32),
            pltpu.SemaphoreType.DMA((_QDEPTH,)),
        ],
        compiler_params=pltpu.CompilerParams(
            dimension_semantics=("arbitrary",),
            vmem_limit_bytes=66584576,
        ),
        name="qlora_fused",
    )(jnp.array(_NF4_VALS, dtype=jnp.float32).reshape(1, 16),
      scales,
      jnp.repeat(jnp.eye(scales.shape[1], dtype=jnp.float32), _QBLOCK, axis=1),
      lora_a, lora_b, w_idx, x2)

    return y2.reshape(b, s, o)


# speedup vs baseline: 1.0070x; 1.0070x over previous
import jax
import jax.numpy as jnp
from jax.experimental import pallas as pl
from jax.experimental.pallas import tpu as pltpu

_NF4_VALS = (
    -1.0, -0.6961928009986877, -0.5250730514526367, -0.39491748809814453,
    -0.28444138169288635, -0.18477343022823334, -0.09105003625154495, 0.0,
    0.07958029955625534, 0.16093020141124725, 0.24611230194568634,
    0.33791524171829224, 0.44070982933044434, 0.5626170039176941,
    0.7229568362236023, 1.0)

_QBLOCK = 64
_LORA_SCALE = 2.0
_NSLAB = 8
_QDEPTH = 4


def _fused_kernel(cb_ref, scales_ref, sexp_ref, lora_a_ref, lora_b_ref,
                  w_idx_ref, x_ref, y_ref, w_eff_ref, wbuf_ref, sem_ref):
    i = pl.program_id(0)

    @pl.when(i == 0)
    def _prep():
        o, d = w_eff_ref.shape
        rows = o // _NSLAB

        def copy(j):
            return pltpu.make_async_copy(
                w_idx_ref.at[pl.ds(j * rows, rows), :],
                wbuf_ref.at[j % _QDEPTH], sem_ref.at[j % _QDEPTH])

        for j in range(_QDEPTH):
            copy(j).start()
        for j in range(_NSLAB):
            copy(j).wait()
            idx = wbuf_ref[j % _QDEPTH]
            cb = jnp.broadcast_to(cb_ref[...], (rows, 16))
            deq = jnp.take_along_axis(cb, idx, axis=1)
            sc = jax.lax.dot_general(
                scales_ref[pl.ds(j * rows, rows), :], sexp_ref[...],
                (((1,), (0,)), ((), ())), preferred_element_type=jnp.float32)
            lora = jax.lax.dot_general(
                lora_b_ref[pl.ds(j * rows, rows), :], lora_a_ref[...],
                (((1,), (0,)), ((), ())), preferred_element_type=jnp.float32)
            w_eff_ref[pl.ds(j * rows, rows), :] = (
                deq * sc + _LORA_SCALE * lora).astype(jnp.bfloat16)
            if j + _QDEPTH < _NSLAB:
                copy(j + _QDEPTH).start()

    @pl.when(i > 0)
    def _mm():
        xb = x_ref[...].astype(jnp.bfloat16)
        y_ref[...] = jax.lax.dot_general(
            xb, w_eff_ref[...], (((1,), (1,)), ((), ())),
            preferred_element_type=jnp.float32)


def kernel(x, w_idx, scales, lora_a, lora_b):
    b, s, d = x.shape
    o = w_idx.shape[0]
    m = b * s
    x2 = x.reshape(m, d)

    mt = 2048
    nsteps = m // mt

    def xmap(i):
        return (jnp.maximum(i - 1, 0), 0)

    y2 = pl.pallas_call(
        _fused_kernel,
        grid=(nsteps + 1,),
        in_specs=[
            pl.BlockSpec((1, 16), lambda i: (0, 0)),
            pl.BlockSpec((o, scales.shape[1]), lambda i: (0, 0)),
            pl.BlockSpec((scales.shape[1], d), lambda i: (0, 0)),
            pl.BlockSpec(lora_a.shape, lambda i: (0, 0)),
            pl.BlockSpec((o, lora_b.shape[1]), lambda i: (0, 0)),
            pl.BlockSpec(memory_space=pl.ANY),
            pl.BlockSpec((mt, d), xmap),
        ],
        out_specs=pl.BlockSpec((mt, o), xmap),
        out_shape=jax.ShapeDtypeStruct((m, o), jnp.float32),
        scratch_shapes=[
            pltpu.VMEM((o, d), jnp.bfloat16),
            pltpu.VMEM((_QDEPTH, o // _NSLAB, d), jnp.int32),
            pltpu.SemaphoreType.DMA((_QDEPTH,)),
        ],
        compiler_params=pltpu.CompilerParams(
            dimension_semantics=("arbitrary",),
            vmem_limit_bytes=66584576,
        ),
        name="qlora_fused",
    )(jnp.array(_NF4_VALS, dtype=jnp.float32).reshape(1, 16),
      scales,
      jnp.repeat(jnp.eye(scales.shape[1], dtype=jnp.float32), _QBLOCK, axis=1),
      lora_a, lora_b, w_idx, x2)

    return y2.reshape(b, s, o)
